# baseline (speedup 1.0000x reference)
import functools

import jax
import jax.numpy as jnp
from jax import lax
from jax.experimental import pallas as pl
from jax.experimental.pallas import tpu as pltpu
from jax.experimental.pallas import tpu_sc as plsc

B = 16384
S = 50
D = 32

NC = 2
NS = 16
NW = NC * NS
BBLK = 512
NBB = B // BBLK
UNITS = S * NBB
UNITS_PER_W = UNITS // NW

_mesh = plsc.VectorSubcoreMesh(core_axis_name="c", subcore_axis_name="s")


@functools.partial(
    pl.kernel,
    mesh=_mesh,
    out_type=jax.ShapeDtypeStruct((S, D, B), jnp.float32),
    scratch_types=[
        pltpu.VMEM((BBLK,), jnp.int32),
        pltpu.VMEM((BBLK,), jnp.int32),
        pltpu.VMEM((BBLK, D), jnp.float32),
        pltpu.VMEM((BBLK, D), jnp.float32),
        pltpu.VMEM((D, BBLK), jnp.float32),
        pltpu.VMEM((D, BBLK), jnp.float32),
        pltpu.SemaphoreType.DMA,
        pltpu.SemaphoreType.DMA,
        pltpu.SemaphoreType.DMA,
        pltpu.SemaphoreType.DMA,
    ],
    compiler_params=pltpu.CompilerParams(use_tc_tiling_on_sc=False,
                                         needs_layout_passes=False),
)
def _gather_kernel(idx_hbm, table_hbm, out_hbm,
                   idx0, idx1, g0, g1, t0, t1,
                   gsem0, gsem1, ssem0, ssem1):
    idxs = (idx0, idx1)
    gs = (g0, g1)
    ts = (t0, t1)
    gsems = (gsem0, gsem1)
    ssems = (ssem0, ssem1)

    wid = lax.axis_index("s") * NC + lax.axis_index("c")
    base_rows = lax.iota(jnp.int32, 16)

    def s_bb(k):
        u = wid * UNITS_PER_W + k
        return u // NBB, u % NBB

    def load_idx(k, v):
        s, bb = s_bb(k)
        pltpu.sync_copy(idx_hbm.at[s, pl.ds(bb * BBLK, BBLK)], idxs[v])

    def gather_desc(v):
        return pltpu.make_async_copy(table_hbm.at[idxs[v]], gs[v], gsems[v])

    def store_desc(k, v):
        s, bb = s_bb(k)
        return pltpu.make_async_copy(
            ts[v], out_hbm.at[s, :, pl.ds(bb * BBLK, BBLK)], ssems[v])

    def transpose(v):
        g_v, t_v = gs[v], ts[v]

        @plsc.parallel_loop(0, BBLK // 16, unroll=4)
        def tbody(g):
            rows = base_rows + g * 16
            vals = [plsc.load_gather(g_v, [rows, jnp.full((16,), d, jnp.int32)])
                    for d in range(D)]
            for d in range(D):
                t_v[d, pl.ds(g * 16, 16)] = vals[d]

    load_idx(0, 0)
    gather_desc(0).start()

    def pair(p, carry):
        for v in (0, 1):
            k = 2 * p + v
            nk = k + 1
            gather_desc(v).wait()

            @pl.when(nk < UNITS_PER_W)
            def _():
                load_idx(nk, 1 - v)
                gather_desc(1 - v).start()

            @pl.when(k >= 2)
            def _():
                store_desc(k - 2, v).wait()

            transpose(v)
            store_desc(k, v).start()
        return carry

    lax.fori_loop(0, UNITS_PER_W // 2, pair, 0)

    store_desc(UNITS_PER_W - 2, 0).wait()
    store_desc(UNITS_PER_W - 1, 1).wait()


def kernel(inputs, table):
    idx_t = inputs.T.astype(jnp.int32)
    out_t = _gather_kernel(idx_t, table)
    return jnp.transpose(out_t, (2, 0, 1))


# speedup vs baseline: 1.2180x; 1.2180x over previous
import functools

import jax
import jax.numpy as jnp
from jax import lax
from jax.experimental import pallas as pl
from jax.experimental.pallas import tpu as pltpu
from jax.experimental.pallas import tpu_sc as plsc

B = 16384
S = 50
D = 32

NC = 2
NS = 16
NW = NC * NS
BBLK = 512
NBB = B // BBLK
UNITS = S * NBB
UNITS_PER_W = UNITS // NW

_mesh = plsc.VectorSubcoreMesh(core_axis_name="c", subcore_axis_name="s")


@functools.partial(
    pl.kernel,
    mesh=_mesh,
    out_type=jax.ShapeDtypeStruct((S, D, B), jnp.float32),
    scratch_types=[
        pltpu.VMEM((BBLK,), jnp.int32),
        pltpu.VMEM((BBLK,), jnp.int32),
        pltpu.VMEM((BBLK, D), jnp.float32),
        pltpu.VMEM((BBLK, D), jnp.float32),
        pltpu.VMEM((D, BBLK), jnp.float32),
        pltpu.VMEM((D, BBLK), jnp.float32),
        pltpu.SemaphoreType.DMA,
        pltpu.SemaphoreType.DMA,
        pltpu.SemaphoreType.DMA,
        pltpu.SemaphoreType.DMA,
    ],
    compiler_params=pltpu.CompilerParams(use_tc_tiling_on_sc=False,
                                         needs_layout_passes=False),
)
def _gather_kernel(idx_hbm, table_hbm, out_hbm,
                   idx0, idx1, g0, g1, t0, t1,
                   gsem0, gsem1, ssem0, ssem1):
    idxs = (idx0, idx1)
    gs = (g0, g1)
    ts = (t0, t1)
    gsems = (gsem0, gsem1)
    ssems = (ssem0, ssem1)

    wid = lax.axis_index("s") * NC + lax.axis_index("c")
    base_rows = lax.iota(jnp.int32, 16)

    def s_bb(k):
        u = wid * UNITS_PER_W + k
        return u // NBB, u % NBB

    def load_idx(k, v):
        s, bb = s_bb(k)
        pltpu.sync_copy(idx_hbm.at[s, pl.ds(bb * BBLK, BBLK)], idxs[v])

    def gather_desc(v):
        return pltpu.make_async_copy(table_hbm.at[idxs[v]], gs[v], gsems[v])

    def store_desc(k, v):
        s, bb = s_bb(k)
        return pltpu.make_async_copy(
            ts[v], out_hbm.at[s, :, pl.ds(bb * BBLK, BBLK)], ssems[v])

    def transpose(v):
        g_v, t_v = gs[v], ts[v]

        @plsc.parallel_loop(0, BBLK // 16, unroll=2)
        def tbody(g):
            rows = base_rows + g * 16
            vals = [plsc.load_gather(g_v, [rows, jnp.full((16,), d, jnp.int32)])
                    for d in range(D)]
            for d in range(D):
                t_v[d, pl.ds(g * 16, 16)] = vals[d]

    load_idx(0, 0)
    gather_desc(0).start()

    def pair(p, carry):
        for v in (0, 1):
            k = 2 * p + v
            nk = k + 1
            gather_desc(v).wait()

            @pl.when(nk < UNITS_PER_W)
            def _():
                load_idx(nk, 1 - v)
                gather_desc(1 - v).start()

            @pl.when(k >= 2)
            def _():
                store_desc(k - 2, v).wait()

            transpose(v)
            store_desc(k, v).start()
        return carry

    lax.fori_loop(0, UNITS_PER_W // 2, pair, 0)

    store_desc(UNITS_PER_W - 2, 0).wait()
    store_desc(UNITS_PER_W - 1, 1).wait()


def kernel(inputs, table):
    idx_t = inputs.T.astype(jnp.int32)
    out_t = _gather_kernel(idx_t, table)
    return jnp.transpose(out_t, (2, 0, 1))
